# Initial kernel scaffold; baseline (speedup 1.0000x reference)
import jax
import jax.numpy as jnp
from jax.experimental import pallas as pl


def kernel(h, edge_index, edge_weight, g0, b0, Wp1, bp1, Wl1, bl1, Wr1, Wp2, bp2, Wl2, bl2, Wr2, Wp3, bp3, Wl3, bl3, Wr3):
    raise NotImplementedError("write your pallas kernel here")


# speedup vs baseline: 5.3781x; 5.3781x over previous
import functools

import jax
import jax.numpy as jnp
from jax import lax
from jax.experimental import pallas as pl
from jax.experimental.pallas import tpu as pltpu
from jax.experimental.pallas import tpu_sc as plsc

NC = 2
NS = 16
L = 16

CH = 512
SUB = 128
R = 512
D16 = 16


def _pad_rows(a, rows):
    return jnp.concatenate(
        [a, jnp.zeros((rows - a.shape[0],) + a.shape[1:], a.dtype)], axis=0)


def _bn(h, g0, b0):
    return (h * (1.0 / (1.0 + 1e-5) ** 0.5)) * g0 + b0


def _norm_div(o):
    nrm = jnp.maximum(jnp.sqrt(jnp.sum(o * o, axis=1, keepdims=True)), 1e-12)
    return o / nrm


def _tc_a(h, g0, b0, wp1, bp1, xp1t):
    x = _bn(h[...], g0[...], b0[...])
    xp = jax.nn.relu(jnp.dot(x, wp1[...], preferred_element_type=jnp.float32)
                     + bp1[...])
    one = jnp.ones((R, 1), jnp.float32)
    zer = jnp.zeros((R, D16 - 6), jnp.float32)
    xp1t[...] = jnp.concatenate([xp, one, zer], axis=1)


def _tc_b(h, s1a, s1b, g0, b0, wl1, bl1, wr1, wp2, bp2, x2o, xp2o):
    x = _bn(h[...], g0[...], b0[...])
    s = s1a[...] + s1b[...]
    cntc = jnp.maximum(s[:, 5:6], 1.0)
    agg = s[:, 0:5] / cntc
    o = (jnp.dot(agg, wl1[...], preferred_element_type=jnp.float32) + bl1[...]
         + jnp.dot(x, wr1[...], preferred_element_type=jnp.float32))
    x2 = jax.nn.relu(_norm_div(o))
    x2o[...] = x2
    xp2o[...] = jax.nn.relu(
        jnp.dot(x2, wp2[...], preferred_element_type=jnp.float32) + bp2[...])


def _tc_c(s2q0, s2q1, s2q2, s2q3, s1a, s1b, x2, wl2, bl2, wr2, wp3, bp3, wl3,
          x3o, y3o):
    wl2v = wl2[...]
    acc = jnp.dot(s2q0[...], wl2v[0:16, :], preferred_element_type=jnp.float32)
    acc += jnp.dot(s2q1[...], wl2v[16:32, :], preferred_element_type=jnp.float32)
    acc += jnp.dot(s2q2[...], wl2v[32:48, :], preferred_element_type=jnp.float32)
    acc += jnp.dot(s2q3[...], wl2v[48:64, :], preferred_element_type=jnp.float32)
    s = s1a[...] + s1b[...]
    cntc = jnp.maximum(s[:, 5:6], 1.0)
    o = (acc / cntc + bl2[...]
         + jnp.dot(x2[...], wr2[...], preferred_element_type=jnp.float32))
    x3 = jax.nn.relu(_norm_div(o))
    x3o[...] = x3
    xp3 = jax.nn.relu(
        jnp.dot(x3, wp3[...], preferred_element_type=jnp.float32) + bp3[...])
    y3 = jnp.dot(xp3, wl3[...], preferred_element_type=jnp.float32)
    y3o[...] = jnp.concatenate(
        [y3, jnp.zeros((R, D16 - 5), jnp.float32)], axis=1)


def _tc_d(s3a, s3b, s1a, s1b, x3, wr3, bl3, out):
    s3 = s3a[...] + s3b[...]
    s = s1a[...] + s1b[...]
    cntc = jnp.maximum(s[:, 5:6], 1.0)
    agg = s3[:, 0:5] / cntc
    o = (agg + bl3[...]
         + jnp.dot(x3[...], wr3[...], preferred_element_type=jnp.float32))
    out[...] = _norm_div(o)


def _row_spec(off_blocks=0):
    return pl.BlockSpec((R, D16), lambda i, o=off_blocks: (o + i, 0))


def _full(shape):
    return pl.BlockSpec(shape, lambda i: tuple(0 for _ in shape))


_GDN = lax.GatherDimensionNumbers(
    offset_dims=(), collapsed_slice_dims=(0,), start_index_map=(0,))


def _bcast_lane(vec, k):
    idx = jnp.full((L, 1), k, jnp.int32)
    return lax.gather(vec, idx, _GDN, (1,),
                      mode=lax.GatherScatterMode.PROMISE_IN_BOUNDS)


def _scale_chunk(g_ref, ew_ref, with_cnt):
    lane = lax.iota(jnp.int32, L)

    def group(g, _):
        base = g * L
        wv = ew_ref[pl.ds(base, L)]

        for k in range(L):
            j = base + k
            w = _bcast_lane(wv, k)
            if with_cnt:
                w = jnp.where(lane == 5, 1.0, w)
            g_ref[j] = g_ref[j] * w
        return ()

    lax.fori_loop(0, CH // L, group, ())


def _sc_edge_pass_13(nacc, rpt, n_chunks, with_cnt):
    def body(src_h, dst_h, ew_h, tab_h, z_h, out_h, idxb, dstb, ewb, g, acc,
             sem):
        c = lax.axis_index("c")
        s = lax.axis_index("s")
        row0 = s * rpt
        pltpu.sync_copy(z_h, acc.at[pl.ds(row0, rpt)])
        plsc.subcore_barrier()
        ebase = (c * NS + s) * (n_chunks * CH)

        def chunk(i, _):
            off = ebase + i * CH
            for u in range(4):
                pltpu.sync_copy(src_h.at[pl.ds(off + u * SUB, SUB)],
                                idxb.at[u])
            cps = [pltpu.async_copy(tab_h.at[idxb.at[u]],
                                    g.at[pl.ds(u * SUB, SUB), :], sem)
                   for u in range(4)]
            pltpu.sync_copy(ew_h.at[pl.ds(off, CH)], ewb)
            for u in range(4):
                pltpu.sync_copy(dst_h.at[pl.ds(off + u * SUB, SUB)],
                                dstb.at[u])
            for cp in cps:
                cp.wait()
            _scale_chunk(g, ewb, with_cnt)
            for u in range(4):
                pltpu.sync_copy(g.at[pl.ds(u * SUB, SUB), :],
                                acc.at[dstb.at[u]], add=True)
            return ()

        lax.fori_loop(0, n_chunks, chunk, ())
        plsc.subcore_barrier()
        pltpu.sync_copy(acc.at[pl.ds(row0, rpt)],
                        out_h.at[pl.ds(c * nacc + row0, rpt)])

    return body


def _sc_edge_pass_2(nacc, rpt, n_chunks):
    def body(src_h, dst_h, ew_h, tab_h, z_h, out_h, idxb, gidx, dstb, ewb, g,
             acc, sem):
        c = lax.axis_index("c")
        s = lax.axis_index("s")
        row0 = s * rpt
        ebase = s * (n_chunks * CH)

        for p in range(2):
            q = c * 2 + p
            pltpu.sync_copy(z_h, acc.at[pl.ds(row0, rpt)])
            plsc.subcore_barrier()

            def chunk(i, _):
                off = ebase + i * CH
                for u in range(4):
                    pltpu.sync_copy(src_h.at[pl.ds(off + u * SUB, SUB)],
                                    idxb.at[u])
                for u in range(4):
                    r = idxb.at[u]
                    ro = gidx.at[u]
                    for k in range(SUB // L):
                        v = r[pl.ds(k * L, L)]
                        ro[pl.ds(k * L, L)] = v * 4 + q
                cps = [pltpu.async_copy(tab_h.at[gidx.at[u]],
                                        g.at[pl.ds(u * SUB, SUB), :], sem)
                       for u in range(4)]
                pltpu.sync_copy(ew_h.at[pl.ds(off, CH)], ewb)
                for u in range(4):
                    pltpu.sync_copy(dst_h.at[pl.ds(off + u * SUB, SUB)],
                                    dstb.at[u])
                for cp in cps:
                    cp.wait()
                _scale_chunk(g, ewb, False)
                for u in range(4):
                    pltpu.sync_copy(g.at[pl.ds(u * SUB, SUB), :],
                                    acc.at[dstb.at[u]], add=True)
                return ()

            lax.fori_loop(0, n_chunks, chunk, ())
            plsc.subcore_barrier()
            pltpu.sync_copy(acc.at[pl.ds(row0, rpt)],
                            out_h.at[pl.ds(q * nacc + row0, rpt)])
            plsc.subcore_barrier()

    return body


def kernel(h, edge_index, edge_weight, g0, b0, Wp1, bp1, Wl1, bl1, Wr1,
           Wp2, bp2, Wl2, bl2, Wr2, Wp3, bp3, Wl3, bl3, Wr3):
    n = h.shape[0]
    e = edge_weight.shape[0]
    f32 = jnp.float32

    npad = -(-n // R) * R
    rpt = npad // NS
    dump = npad - n
    nblk = npad // R

    epad = -(-e // (NC * NS * CH)) * (NC * NS * CH)
    nch13 = epad // (NC * NS * CH)
    nch2 = epad // (NS * CH)
    pe = epad - e
    pidx = lax.iota(jnp.int32, pe) if pe else jnp.zeros((0,), jnp.int32)
    src = jnp.concatenate([edge_index[0], pidx % n])
    dst = jnp.concatenate([edge_index[1], n + pidx % max(dump, 1)])
    ew = jnp.concatenate([edge_weight, jnp.zeros((pe,), f32)])

    hp = _pad_rows(h, npad)
    zrows = jnp.zeros((rpt, D16), f32)
    g0r, b0r = g0.reshape(1, 5), b0.reshape(1, 5)
    bp1r, bl1r = bp1.reshape(1, 5), bl1.reshape(1, 64)
    bp2r, bl2r = bp2.reshape(1, 64), bl2.reshape(1, 64)
    bp3r, bl3r = bp3.reshape(1, 64), bl3.reshape(1, 5)

    mesh = plsc.VectorSubcoreMesh(core_axis_name="c", subcore_axis_name="s",
                                  num_cores=NC, num_subcores=NS)
    sc13_scratch = [
        pltpu.VMEM((4, SUB), jnp.int32),
        pltpu.VMEM((4, SUB), jnp.int32),
        pltpu.VMEM((CH,), f32),
        pltpu.VMEM((CH, D16), f32),
        pltpu.VMEM_SHARED((npad, D16), f32),
        pltpu.SemaphoreType.DMA,
    ]
    sc2_scratch = [
        pltpu.VMEM((4, SUB), jnp.int32),
        pltpu.VMEM((4, SUB), jnp.int32),
        pltpu.VMEM((4, SUB), jnp.int32),
        pltpu.VMEM((CH,), f32),
        pltpu.VMEM((CH, D16), f32),
        pltpu.VMEM_SHARED((npad, D16), f32),
        pltpu.SemaphoreType.DMA,
    ]

    sc_params = pltpu.CompilerParams(use_tc_tiling_on_sc=False)
    sc1 = pl.kernel(_sc_edge_pass_13(npad, rpt, nch13, True),
                    out_type=jax.ShapeDtypeStruct((2 * npad, D16), f32),
                    mesh=mesh, scratch_types=sc13_scratch,
                    compiler_params=sc_params)
    sc3 = pl.kernel(_sc_edge_pass_13(npad, rpt, nch13, False),
                    out_type=jax.ShapeDtypeStruct((2 * npad, D16), f32),
                    mesh=mesh, scratch_types=sc13_scratch,
                    compiler_params=sc_params)
    sc2 = pl.kernel(_sc_edge_pass_2(npad, rpt, nch2),
                    out_type=jax.ShapeDtypeStruct((4 * npad, D16), f32),
                    mesh=mesh, scratch_types=sc2_scratch,
                    compiler_params=sc_params)

    xp1t = pl.pallas_call(
        _tc_a,
        grid=(nblk,),
        in_specs=[pl.BlockSpec((R, 5), lambda i: (i, 0)), _full((1, 5)),
                  _full((1, 5)), _full((5, 5)), _full((1, 5))],
        out_specs=_row_spec(),
        out_shape=jax.ShapeDtypeStruct((npad, D16), f32),
    )(hp, g0r, b0r, Wp1, bp1r)

    s1 = sc1(src, dst, ew, xp1t, zrows)

    x2, xp2 = pl.pallas_call(
        _tc_b,
        grid=(nblk,),
        in_specs=[pl.BlockSpec((R, 5), lambda i: (i, 0)),
                  _row_spec(0), _row_spec(nblk),
                  _full((1, 5)), _full((1, 5)), _full((5, 64)),
                  _full((1, 64)), _full((5, 64)), _full((64, 64)),
                  _full((1, 64))],
        out_specs=[pl.BlockSpec((R, 64), lambda i: (i, 0)),
                   pl.BlockSpec((R, 64), lambda i: (i, 0))],
        out_shape=[jax.ShapeDtypeStruct((npad, 64), f32),
                   jax.ShapeDtypeStruct((npad, 64), f32)],
    )(hp, s1, s1, g0r, b0r, Wl1, bl1r, Wr1, Wp2, bp2r)

    s2 = sc2(src, dst, ew, xp2.reshape(4 * npad, D16), zrows)

    x3, y3t = pl.pallas_call(
        _tc_c,
        grid=(nblk,),
        in_specs=[_row_spec(0), _row_spec(nblk), _row_spec(2 * nblk),
                  _row_spec(3 * nblk), _row_spec(0), _row_spec(nblk),
                  pl.BlockSpec((R, 64), lambda i: (i, 0)),
                  _full((64, 64)), _full((1, 64)), _full((64, 64)),
                  _full((64, 64)), _full((1, 64)), _full((64, 5))],
        out_specs=[pl.BlockSpec((R, 64), lambda i: (i, 0)), _row_spec()],
        out_shape=[jax.ShapeDtypeStruct((npad, 64), f32),
                   jax.ShapeDtypeStruct((npad, D16), f32)],
    )(s2, s2, s2, s2, s1, s1, x2, Wl2, bl2r, Wr2, Wp3, bp3r, Wl3)

    s3 = sc3(src, dst, ew, y3t, zrows)

    out = pl.pallas_call(
        _tc_d,
        grid=(nblk,),
        in_specs=[_row_spec(0), _row_spec(nblk), _row_spec(0),
                  _row_spec(nblk),
                  pl.BlockSpec((R, 64), lambda i: (i, 0)),
                  _full((64, 5)), _full((1, 5))],
        out_specs=pl.BlockSpec((R, 5), lambda i: (i, 0)),
        out_shape=jax.ShapeDtypeStruct((npad, 5), f32),
    )(s3, s3, s1, s1, x3, Wr3, bl3r)

    return out[:n]


# speedup vs baseline: 9.8591x; 1.8332x over previous
import functools

import jax
import jax.numpy as jnp
from jax import lax
from jax.experimental import pallas as pl
from jax.experimental.pallas import tpu as pltpu
from jax.experimental.pallas import tpu_sc as plsc

NC = 2
NS = 16
L = 16

CH = 512
SUB = 128
NSUB = CH // SUB
R = 512
D16 = 16
PKROWS = 2 * NSUB


def _pad_rows(a, rows):
    return jnp.concatenate(
        [a, jnp.zeros((rows - a.shape[0],) + a.shape[1:], a.dtype)], axis=0)


def _bn(h, g0, b0):
    return (h * (1.0 / (1.0 + 1e-5) ** 0.5)) * g0 + b0


def _norm_div(o):
    nrm = jnp.maximum(jnp.sqrt(jnp.sum(o * o, axis=1, keepdims=True)), 1e-12)
    return o / nrm


def _tc_a(h, g0, b0, wp1, bp1, xp1t):
    x = _bn(h[...], g0[...], b0[...])
    xp = jax.nn.relu(jnp.dot(x, wp1[...], preferred_element_type=jnp.float32)
                     + bp1[...])
    one = jnp.ones((R, 1), jnp.float32)
    zer = jnp.zeros((R, D16 - 6), jnp.float32)
    xp1t[...] = jnp.concatenate([xp, one, zer], axis=1)


def _tc_b(h, s1a, s1b, g0, b0, wl1, bl1, wr1, wp2, bp2, x2o, xp2o):
    x = _bn(h[...], g0[...], b0[...])
    s = s1a[...] + s1b[...]
    cntc = jnp.maximum(s[:, 5:6], 1.0)
    agg = s[:, 0:5] / cntc
    o = (jnp.dot(agg, wl1[...], preferred_element_type=jnp.float32) + bl1[...]
         + jnp.dot(x, wr1[...], preferred_element_type=jnp.float32))
    x2 = jax.nn.relu(_norm_div(o))
    x2o[...] = x2
    xp2o[...] = jax.nn.relu(
        jnp.dot(x2, wp2[...], preferred_element_type=jnp.float32) + bp2[...])


def _tc_c(s2q0, s2q1, s2q2, s2q3, s1a, s1b, x2, wl2, bl2, wr2, wp3, bp3, wl3,
          x3o, y3o):
    wl2v = wl2[...]
    acc = jnp.dot(s2q0[...], wl2v[0:16, :], preferred_element_type=jnp.float32)
    acc += jnp.dot(s2q1[...], wl2v[16:32, :], preferred_element_type=jnp.float32)
    acc += jnp.dot(s2q2[...], wl2v[32:48, :], preferred_element_type=jnp.float32)
    acc += jnp.dot(s2q3[...], wl2v[48:64, :], preferred_element_type=jnp.float32)
    s = s1a[...] + s1b[...]
    cntc = jnp.maximum(s[:, 5:6], 1.0)
    o = (acc / cntc + bl2[...]
         + jnp.dot(x2[...], wr2[...], preferred_element_type=jnp.float32))
    x3 = jax.nn.relu(_norm_div(o))
    x3o[...] = x3
    xp3 = jax.nn.relu(
        jnp.dot(x3, wp3[...], preferred_element_type=jnp.float32) + bp3[...])
    y3 = jnp.dot(xp3, wl3[...], preferred_element_type=jnp.float32)
    y3o[...] = jnp.concatenate(
        [y3, jnp.zeros((R, D16 - 5), jnp.float32)], axis=1)


def _tc_d(s3a, s3b, s1a, s1b, x3, wr3, bl3, out):
    s3 = s3a[...] + s3b[...]
    s = s1a[...] + s1b[...]
    cntc = jnp.maximum(s[:, 5:6], 1.0)
    agg = s3[:, 0:5] / cntc
    o = (agg + bl3[...]
         + jnp.dot(x3[...], wr3[...], preferred_element_type=jnp.float32))
    out[...] = _norm_div(o)


def _row_spec(off_blocks=0):
    return pl.BlockSpec((R, D16), lambda i, o=off_blocks: (o + i, 0))


def _full(shape):
    return pl.BlockSpec(shape, lambda i: tuple(0 for _ in shape))


_GDN = lax.GatherDimensionNumbers(
    offset_dims=(), collapsed_slice_dims=(0,), start_index_map=(0,))


def _bcast_lane(vec, k):
    idx = jnp.full((L, 1), k, jnp.int32)
    return lax.gather(vec, idx, _GDN, (1,),
                      mode=lax.GatherScatterMode.PROMISE_IN_BOUNDS)


def _scale_chunk(g_ref, ew_ref, with_cnt):
    lane = lax.iota(jnp.int32, L)

    def group(g, _):
        base = g * L
        wv = ew_ref[pl.ds(base, L)]

        for k in range(L):
            j = base + k
            w = _bcast_lane(wv, k)
            if with_cnt:
                w = jnp.where(lane == 5, 1.0, w)
            g_ref[j] = g_ref[j] * w
        return ()

    lax.fori_loop(0, CH // L, group, (), unroll=2)


def _pipelined_accumulate(epk, ewp, tab, pkb, gidx, sidx, ewb, g, acc,
                          esems, gsems, ssems, cbase, n_chunks, q, with_cnt):
    def start_edges(ci, b):
        pltpu.async_copy(epk.at[ci], pkb.at[b], esems[b])
        pltpu.async_copy(ewp.at[ci], ewb.at[b], esems[b])

    def edges_wait(b):
        pltpu.make_async_copy(epk.at[cbase], pkb.at[b], esems[b]).wait()
        pltpu.make_async_copy(ewp.at[cbase], ewb.at[b], esems[b]).wait()

    def gather_idx_ref(b, u):
        return gidx.at[b, u] if q is not None else pkb.at[b, u]

    def gather_desc(b, u):
        return pltpu.make_async_copy(
            tab.at[gather_idx_ref(b, u)],
            g.at[b, pl.ds(u * SUB, SUB), :], gsems[b])

    def scatter_desc(b, u):
        return pltpu.make_async_copy(
            g.at[b, pl.ds(u * SUB, SUB), :],
            acc.at[sidx.at[b, u]], ssems[b])

    start_edges(cbase, 0)

    def pair(ii, _):
        for b in range(2):
            i = ii * 2 + b
            edges_wait(b)
            start_edges(cbase + i + 1, b ^ 1)
            @pl.when(i >= 2)
            def _():
                for u in range(NSUB):
                    scatter_desc(b, u).wait()
            if q is not None:
                for u in range(NSUB):
                    sr = pkb.at[b, u]
                    gr = gidx.at[b, u]
                    for k in range(SUB // L):
                        gr[pl.ds(k * L, L)] = sr[pl.ds(k * L, L)] * 4 + q
            for u in range(NSUB):
                pltpu.async_copy(tab.at[gather_idx_ref(b, u)],
                                 g.at[b, pl.ds(u * SUB, SUB), :], gsems[b])
            for u in range(NSUB):
                dr = pkb.at[b, NSUB + u]
                sr = sidx.at[b, u]
                for k in range(SUB // L):
                    sr[pl.ds(k * L, L)] = dr[pl.ds(k * L, L)]
            for u in range(NSUB):
                gather_desc(b, u).wait()
            _scale_chunk(g.at[b], ewb.at[b], with_cnt)
            for u in range(NSUB):
                pltpu.async_copy(g.at[b, pl.ds(u * SUB, SUB), :],
                                 acc.at[sidx.at[b, u]], ssems[b], add=True)
        return ()

    lax.fori_loop(0, n_chunks // 2, pair, ())
    for b in range(2):
        for u in range(NSUB):
            scatter_desc(b, u).wait()
    edges_wait(n_chunks % 2)


def _sc_edge_pass_13(nacc, rpt, n_chunks, with_cnt):
    def body(epk, ewp, tab_h, z_h, out_h, pkb, sidx, ewb, g, acc,
             esem0, esem1, gsem0, gsem1, ssem0, ssem1):
        c = lax.axis_index("c")
        s = lax.axis_index("s")
        row0 = s * rpt
        pltpu.sync_copy(z_h, acc.at[pl.ds(row0, rpt)])
        plsc.subcore_barrier()
        cbase = (c * NS + s) * n_chunks
        _pipelined_accumulate(epk, ewp, tab_h, pkb, None, sidx, ewb, g, acc,
                              (esem0, esem1), (gsem0, gsem1), (ssem0, ssem1),
                              cbase, n_chunks, None, with_cnt)
        plsc.subcore_barrier()
        pltpu.sync_copy(acc.at[pl.ds(row0, rpt)],
                        out_h.at[pl.ds(c * nacc + row0, rpt)])

    return body


def _sc_edge_pass_2(nacc, rpt, n_chunks):
    def body(epk, ewp, tab_h, z_h, out_h, pkb, gidx, sidx, ewb, g, acc,
             esem0, esem1, gsem0, gsem1, ssem0, ssem1):
        c = lax.axis_index("c")
        s = lax.axis_index("s")
        row0 = s * rpt
        cbase = s * n_chunks

        for p in range(2):
            q = c * 2 + p
            pltpu.sync_copy(z_h, acc.at[pl.ds(row0, rpt)])
            plsc.subcore_barrier()
            _pipelined_accumulate(epk, ewp, tab_h, pkb, gidx, sidx, ewb, g,
                                  acc, (esem0, esem1), (gsem0, gsem1),
                                  (ssem0, ssem1), cbase, n_chunks, q, False)
            plsc.subcore_barrier()
            pltpu.sync_copy(acc.at[pl.ds(row0, rpt)],
                            out_h.at[pl.ds(q * nacc + row0, rpt)])
            plsc.subcore_barrier()

    return body


def kernel(h, edge_index, edge_weight, g0, b0, Wp1, bp1, Wl1, bl1, Wr1,
           Wp2, bp2, Wl2, bl2, Wr2, Wp3, bp3, Wl3, bl3, Wr3):
    n = h.shape[0]
    e = edge_weight.shape[0]
    f32 = jnp.float32

    npad = -(-n // R) * R
    rpt = npad // NS
    dump = npad - n
    nblk = npad // R

    epad = -(-e // (NC * NS * CH)) * (NC * NS * CH)
    nch13 = epad // (NC * NS * CH)
    nch2 = epad // (NS * CH)
    pe = epad - e
    pidx = lax.iota(jnp.int32, pe) if pe else jnp.zeros((0,), jnp.int32)
    src = jnp.concatenate([edge_index[0], pidx % n])
    dst = jnp.concatenate([edge_index[1], n + pidx % max(dump, 1)])
    ew = jnp.concatenate([edge_weight, jnp.zeros((pe,), f32)])
    ncht = epad // CH
    epk = jnp.concatenate([
        src.reshape(ncht, NSUB, SUB),
        dst.reshape(ncht, NSUB, SUB),
    ], axis=1)
    epk = jnp.concatenate(
        [epk, jnp.zeros((1, PKROWS, SUB), jnp.int32)], axis=0)
    ewp = jnp.concatenate(
        [ew.reshape(ncht, CH), jnp.zeros((1, CH), f32)], axis=0)

    hp = _pad_rows(h, npad)
    zrows = jnp.zeros((rpt, D16), f32)
    g0r, b0r = g0.reshape(1, 5), b0.reshape(1, 5)
    bp1r, bl1r = bp1.reshape(1, 5), bl1.reshape(1, 64)
    bp2r, bl2r = bp2.reshape(1, 64), bl2.reshape(1, 64)
    bp3r, bl3r = bp3.reshape(1, 64), bl3.reshape(1, 5)

    mesh = plsc.VectorSubcoreMesh(core_axis_name="c", subcore_axis_name="s",
                                  num_cores=NC, num_subcores=NS)
    sems = [pltpu.SemaphoreType.DMA] * 6
    common = [
        pltpu.VMEM((2, NSUB, SUB), jnp.int32),
        pltpu.VMEM((2, CH), f32),
        pltpu.VMEM((2, CH, D16), f32),
        pltpu.VMEM_SHARED((npad, D16), f32),
    ]
    sc13_scratch = [pltpu.VMEM((2, PKROWS, SUB), jnp.int32)] + common + sems
    sc2_scratch = ([pltpu.VMEM((2, PKROWS, SUB), jnp.int32),
                    pltpu.VMEM((2, NSUB, SUB), jnp.int32)]
                   + common + sems)

    sc_params = pltpu.CompilerParams(use_tc_tiling_on_sc=False)
    sc1 = pl.kernel(_sc_edge_pass_13(npad, rpt, nch13, True),
                    out_type=jax.ShapeDtypeStruct((2 * npad, D16), f32),
                    mesh=mesh, scratch_types=sc13_scratch,
                    compiler_params=sc_params)
    sc3 = pl.kernel(_sc_edge_pass_13(npad, rpt, nch13, False),
                    out_type=jax.ShapeDtypeStruct((2 * npad, D16), f32),
                    mesh=mesh, scratch_types=sc13_scratch,
                    compiler_params=sc_params)
    sc2 = pl.kernel(_sc_edge_pass_2(npad, rpt, nch2),
                    out_type=jax.ShapeDtypeStruct((4 * npad, D16), f32),
                    mesh=mesh, scratch_types=sc2_scratch,
                    compiler_params=sc_params)

    xp1t = pl.pallas_call(
        _tc_a,
        grid=(nblk,),
        in_specs=[pl.BlockSpec((R, 5), lambda i: (i, 0)), _full((1, 5)),
                  _full((1, 5)), _full((5, 5)), _full((1, 5))],
        out_specs=_row_spec(),
        out_shape=jax.ShapeDtypeStruct((npad, D16), f32),
    )(hp, g0r, b0r, Wp1, bp1r)

    s1 = sc1(epk, ewp, xp1t, zrows)

    x2, xp2 = pl.pallas_call(
        _tc_b,
        grid=(nblk,),
        in_specs=[pl.BlockSpec((R, 5), lambda i: (i, 0)),
                  _row_spec(0), _row_spec(nblk),
                  _full((1, 5)), _full((1, 5)), _full((5, 64)),
                  _full((1, 64)), _full((5, 64)), _full((64, 64)),
                  _full((1, 64))],
        out_specs=[pl.BlockSpec((R, 64), lambda i: (i, 0)),
                   pl.BlockSpec((R, 64), lambda i: (i, 0))],
        out_shape=[jax.ShapeDtypeStruct((npad, 64), f32),
                   jax.ShapeDtypeStruct((npad, 64), f32)],
    )(hp, s1, s1, g0r, b0r, Wl1, bl1r, Wr1, Wp2, bp2r)

    s2 = sc2(epk, ewp, xp2.reshape(4 * npad, D16), zrows)

    x3, y3t = pl.pallas_call(
        _tc_c,
        grid=(nblk,),
        in_specs=[_row_spec(0), _row_spec(nblk), _row_spec(2 * nblk),
                  _row_spec(3 * nblk), _row_spec(0), _row_spec(nblk),
                  pl.BlockSpec((R, 64), lambda i: (i, 0)),
                  _full((64, 64)), _full((1, 64)), _full((64, 64)),
                  _full((64, 64)), _full((1, 64)), _full((64, 5))],
        out_specs=[pl.BlockSpec((R, 64), lambda i: (i, 0)), _row_spec()],
        out_shape=[jax.ShapeDtypeStruct((npad, 64), f32),
                   jax.ShapeDtypeStruct((npad, D16), f32)],
    )(s2, s2, s2, s2, s1, s1, x2, Wl2, bl2r, Wr2, Wp3, bp3r, Wl3)

    s3 = sc3(epk, ewp, y3t, zrows)

    out = pl.pallas_call(
        _tc_d,
        grid=(nblk,),
        in_specs=[_row_spec(0), _row_spec(nblk), _row_spec(0),
                  _row_spec(nblk),
                  pl.BlockSpec((R, 64), lambda i: (i, 0)),
                  _full((64, 5)), _full((1, 5))],
        out_specs=pl.BlockSpec((R, 5), lambda i: (i, 0)),
        out_shape=jax.ShapeDtypeStruct((npad, 5), f32),
    )(s3, s3, s1, s1, x3, Wr3, bl3r)

    return out[:n]


# speedup vs baseline: 9.9883x; 1.0131x over previous
import functools

import jax
import jax.numpy as jnp
from jax import lax
from jax.experimental import pallas as pl
from jax.experimental.pallas import tpu as pltpu
from jax.experimental.pallas import tpu_sc as plsc

NC = 2
NS = 16
L = 16

CH = 512
SUB = 128
NSUB = CH // SUB
R = 2048
D16 = 16


def _pad_rows(a, rows):
    return jnp.concatenate(
        [a, jnp.zeros((rows - a.shape[0],) + a.shape[1:], a.dtype)], axis=0)


def _bn(h, g0, b0):
    return (h * (1.0 / (1.0 + 1e-5) ** 0.5)) * g0 + b0


def _norm_div(o):
    nrm = jnp.maximum(jnp.sqrt(jnp.sum(o * o, axis=1, keepdims=True)), 1e-12)
    return o / nrm


def _tc_a(h, g0, b0, wp1, bp1, xp1t):
    x = _bn(h[...], g0[...], b0[...])
    xp = jax.nn.relu(jnp.dot(x, wp1[...], preferred_element_type=jnp.float32)
                     + bp1[...])
    one = jnp.ones((R, 1), jnp.float32)
    zer = jnp.zeros((R, D16 - 6), jnp.float32)
    xp1t[...] = jnp.concatenate([xp, one, zer], axis=1)


def _tc_b(h, s1a, s1b, g0, b0, wl1, bl1, wr1, x2o):
    x = _bn(h[...], g0[...], b0[...])
    s = s1a[...] + s1b[...]
    cntc = jnp.maximum(s[:, 5:6], 1.0)
    agg = s[:, 0:5] / cntc
    o = (jnp.dot(agg, wl1[...], preferred_element_type=jnp.float32) + bl1[...]
         + jnp.dot(x, wr1[...], preferred_element_type=jnp.float32))
    x2o[...] = jax.nn.relu(_norm_div(o))


def _tc_b2(x2, wp2q, bp2q, xp2qo):
    xp2qo[...] = jax.nn.relu(
        jnp.dot(x2[...], wp2q[0], preferred_element_type=jnp.float32)
        + bp2q[0])


def _tc_c(s2q0, s2q1, s2q2, s2q3, s1a, s1b, x2, wl2, bl2, wr2, wp3, bp3, wl3,
          x3o, y3o):
    wl2v = wl2[...]
    acc = jnp.dot(s2q0[...], wl2v[0:16, :], preferred_element_type=jnp.float32)
    acc += jnp.dot(s2q1[...], wl2v[16:32, :], preferred_element_type=jnp.float32)
    acc += jnp.dot(s2q2[...], wl2v[32:48, :], preferred_element_type=jnp.float32)
    acc += jnp.dot(s2q3[...], wl2v[48:64, :], preferred_element_type=jnp.float32)
    s = s1a[...] + s1b[...]
    cntc = jnp.maximum(s[:, 5:6], 1.0)
    o = (acc / cntc + bl2[...]
         + jnp.dot(x2[...], wr2[...], preferred_element_type=jnp.float32))
    x3 = jax.nn.relu(_norm_div(o))
    x3o[...] = x3
    xp3 = jax.nn.relu(
        jnp.dot(x3, wp3[...], preferred_element_type=jnp.float32) + bp3[...])
    y3 = jnp.dot(xp3, wl3[...], preferred_element_type=jnp.float32)
    y3o[...] = jnp.concatenate(
        [y3, jnp.zeros((R, D16 - 5), jnp.float32)], axis=1)


def _tc_d(s3a, s3b, s1a, s1b, x3, wr3, bl3, out):
    s3 = s3a[...] + s3b[...]
    s = s1a[...] + s1b[...]
    cntc = jnp.maximum(s[:, 5:6], 1.0)
    agg = s3[:, 0:5] / cntc
    o = (agg + bl3[...]
         + jnp.dot(x3[...], wr3[...], preferred_element_type=jnp.float32))
    out[...] = _norm_div(o)


def _row_spec(off_blocks=0):
    return pl.BlockSpec((R, D16), lambda i, o=off_blocks: (o + i, 0))


def _full(shape):
    return pl.BlockSpec(shape, lambda i: tuple(0 for _ in shape))


_GDN = lax.GatherDimensionNumbers(
    offset_dims=(), collapsed_slice_dims=(0,), start_index_map=(0,))


def _bcast_lane(vec, k):
    idx = jnp.full((L, 1), k, jnp.int32)
    return lax.gather(vec, idx, _GDN, (1,),
                      mode=lax.GatherScatterMode.PROMISE_IN_BOUNDS)


def _scale_chunk(g_ref, ew_ref, with_cnt):
    lane = lax.iota(jnp.int32, L)

    def group(g, _):
        base = g * L
        wv = ew_ref[pl.ds(base, L)]

        for k in range(L):
            j = base + k
            w = _bcast_lane(wv, k)
            if with_cnt:
                w = jnp.where(lane == 5, 1.0, w)
            g_ref[j] = g_ref[j] * w
        return ()

    lax.fori_loop(0, CH // L, group, (), unroll=2)


def _pipelined_accumulate(srcp, dstp, ewp, tab, sbuf, gidx, sidx, ewb, g, acc,
                          esems, gsems, ssems, cbase, n_chunks, q, with_cnt):
    def start_edges(ci, b):
        off = ci * CH
        pltpu.async_copy(srcp.at[pl.ds(off, CH)], sbuf.at[b], esems[b])
        pltpu.async_copy(ewp.at[pl.ds(off, CH)], ewb.at[b], esems[b])
        for u in range(NSUB):
            pltpu.async_copy(dstp.at[pl.ds(off + u * SUB, SUB)],
                             sidx.at[b, u], esems[b])

    def edges_wait(b):
        pltpu.make_async_copy(srcp.at[pl.ds(0, CH)], sbuf.at[b],
                              esems[b]).wait()
        pltpu.make_async_copy(ewp.at[pl.ds(0, CH)], ewb.at[b],
                              esems[b]).wait()
        for u in range(NSUB):
            pltpu.make_async_copy(dstp.at[pl.ds(0, SUB)], sidx.at[b, u],
                                  esems[b]).wait()

    def gather_idx_ref(b, u):
        return gidx.at[b, u] if q is not None else sbuf.at[b, pl.ds(u * SUB,
                                                                    SUB)]

    def scatter_wait(b):
        for u in range(NSUB):
            pltpu.make_async_copy(g.at[b, pl.ds(u * SUB, SUB), :],
                                  acc.at[sidx.at[b, u]], ssems[b]).wait()

    start_edges(cbase, 0)

    def pair(ii, _):
        for b in range(2):
            i = ii * 2 + b
            edges_wait(b)
            @pl.when(i >= 1)
            def _():
                scatter_wait(b ^ 1)
            start_edges(cbase + i + 1, b ^ 1)
            if q is not None:
                for u in range(NSUB):
                    gr = gidx.at[b, u]
                    for k in range(SUB // L):
                        gr[pl.ds(k * L, L)] = (
                            sbuf.at[b][pl.ds(u * SUB + k * L, L)] + q)
            for u in range(NSUB):
                pltpu.async_copy(tab.at[gather_idx_ref(b, u)],
                                 g.at[b, pl.ds(u * SUB, SUB), :], gsems[b])
            for u in range(NSUB):
                pltpu.make_async_copy(tab.at[gather_idx_ref(b, u)],
                                      g.at[b, pl.ds(u * SUB, SUB), :],
                                      gsems[b]).wait()
            _scale_chunk(g.at[b], ewb.at[b], with_cnt)
            for u in range(NSUB):
                pltpu.async_copy(g.at[b, pl.ds(u * SUB, SUB), :],
                                 acc.at[sidx.at[b, u]], ssems[b], add=True)
        return ()

    lax.fori_loop(0, n_chunks // 2, pair, ())
    scatter_wait((n_chunks - 1) % 2)
    edges_wait(n_chunks % 2)


def _sc_edge_pass_13(nacc, rpt, n_chunks, with_cnt):
    def body(srcp, dstp, ewp, tab_h, z_h, out_h, sbuf, sidx, ewb, g, acc,
             esem0, esem1, gsem0, gsem1, ssem0, ssem1):
        c = lax.axis_index("c")
        s = lax.axis_index("s")
        row0 = s * rpt
        pltpu.sync_copy(z_h, acc.at[pl.ds(row0, rpt)])
        plsc.subcore_barrier()
        cbase = (c * NS + s) * n_chunks
        _pipelined_accumulate(srcp, dstp, ewp, tab_h, sbuf, None, sidx, ewb,
                              g, acc, (esem0, esem1), (gsem0, gsem1),
                              (ssem0, ssem1), cbase, n_chunks, None, with_cnt)
        plsc.subcore_barrier()
        pltpu.sync_copy(acc.at[pl.ds(row0, rpt)],
                        out_h.at[pl.ds(c * nacc + row0, rpt)])

    return body


def _sc_edge_pass_2(nacc, rpt, n_chunks):
    def body(srcp, dstp, ewp, tab_h, z_h, out_h, sbuf, gidx, sidx, ewb, g,
             acc, esem0, esem1, gsem0, gsem1, ssem0, ssem1):
        c = lax.axis_index("c")
        s = lax.axis_index("s")
        row0 = s * rpt
        cbase = s * n_chunks

        for p in range(2):
            q = c * 2 + p
            pltpu.sync_copy(z_h, acc.at[pl.ds(row0, rpt)])
            plsc.subcore_barrier()
            _pipelined_accumulate(srcp, dstp, ewp, tab_h, sbuf, gidx, sidx,
                                  ewb, g, acc, (esem0, esem1), (gsem0, gsem1),
                                  (ssem0, ssem1), cbase, n_chunks, q * nacc,
                                  False)
            plsc.subcore_barrier()
            pltpu.sync_copy(acc.at[pl.ds(row0, rpt)],
                            out_h.at[pl.ds(q * nacc + row0, rpt)])
            plsc.subcore_barrier()

    return body


def kernel(h, edge_index, edge_weight, g0, b0, Wp1, bp1, Wl1, bl1, Wr1,
           Wp2, bp2, Wl2, bl2, Wr2, Wp3, bp3, Wl3, bl3, Wr3):
    n = h.shape[0]
    e = edge_weight.shape[0]
    f32 = jnp.float32

    npad = -(-n // R) * R
    rpt = npad // NS
    dump = npad - n
    nblk = npad // R

    epad = -(-e // (NC * NS * CH)) * (NC * NS * CH)
    nch13 = epad // (NC * NS * CH)
    nch2 = epad // (NS * CH)
    pe = epad + CH - e
    pidx = lax.iota(jnp.int32, pe)
    srcp = jnp.concatenate([edge_index[0], pidx % n])
    dstp = jnp.concatenate([edge_index[1], n + pidx % max(dump, 1)])
    ewp = jnp.concatenate([edge_weight, jnp.zeros((pe,), f32)])

    zrows = jnp.zeros((rpt, D16), f32)
    g0r, b0r = g0.reshape(1, 5), b0.reshape(1, 5)
    bp1r, bl1r = bp1.reshape(1, 5), bl1.reshape(1, 64)
    bp2r, bl2r = bp2.reshape(1, 64), bl2.reshape(1, 64)
    bp3r, bl3r = bp3.reshape(1, 64), bl3.reshape(1, 5)

    mesh = plsc.VectorSubcoreMesh(core_axis_name="c", subcore_axis_name="s",
                                  num_cores=NC, num_subcores=NS)
    sems = [pltpu.SemaphoreType.DMA] * 6
    common = [
        pltpu.VMEM((2, NSUB, SUB), jnp.int32),
        pltpu.VMEM((2, CH), f32),
        pltpu.VMEM((2, CH, D16), f32),
        pltpu.VMEM_SHARED((npad, D16), f32),
    ]
    sc13_scratch = [pltpu.VMEM((2, CH), jnp.int32)] + common + sems
    sc2_scratch = ([pltpu.VMEM((2, CH), jnp.int32),
                    pltpu.VMEM((2, NSUB, SUB), jnp.int32)]
                   + common + sems)

    sc_params = pltpu.CompilerParams(use_tc_tiling_on_sc=False)
    sc1 = pl.kernel(_sc_edge_pass_13(npad, rpt, nch13, True),
                    out_type=jax.ShapeDtypeStruct((2 * npad, D16), f32),
                    mesh=mesh, scratch_types=sc13_scratch,
                    compiler_params=sc_params)
    sc3 = pl.kernel(_sc_edge_pass_13(npad, rpt, nch13, False),
                    out_type=jax.ShapeDtypeStruct((2 * npad, D16), f32),
                    mesh=mesh, scratch_types=sc13_scratch,
                    compiler_params=sc_params)
    sc2 = pl.kernel(_sc_edge_pass_2(npad, rpt, nch2),
                    out_type=jax.ShapeDtypeStruct((4 * npad, D16), f32),
                    mesh=mesh, scratch_types=sc2_scratch,
                    compiler_params=sc_params)

    xp1t = pl.pallas_call(
        _tc_a,
        grid=(nblk,),
        in_specs=[pl.BlockSpec((R, 5), lambda i: (i, 0)), _full((1, 5)),
                  _full((1, 5)), _full((5, 5)), _full((1, 5))],
        out_specs=_row_spec(),
        out_shape=jax.ShapeDtypeStruct((npad, D16), f32),
    )(h, g0r, b0r, Wp1, bp1r)

    s1 = sc1(srcp, dstp, ewp, xp1t, zrows)

    x2 = pl.pallas_call(
        _tc_b,
        grid=(nblk,),
        in_specs=[pl.BlockSpec((R, 5), lambda i: (i, 0)),
                  _row_spec(0), _row_spec(nblk),
                  _full((1, 5)), _full((1, 5)), _full((5, 64)),
                  _full((1, 64)), _full((5, 64))],
        out_specs=pl.BlockSpec((R, 64), lambda i: (i, 0)),
        out_shape=jax.ShapeDtypeStruct((npad, 64), f32),
    )(h, s1, s1, g0r, b0r, Wl1, bl1r, Wr1)

    xp2q = pl.pallas_call(
        _tc_b2,
        grid=(nblk, 4),
        in_specs=[pl.BlockSpec((R, 64), lambda i, f: (i, 0)),
                  pl.BlockSpec((1, 64, D16), lambda i, f: (f, 0, 0)),
                  pl.BlockSpec((1, 1, D16), lambda i, f: (f, 0, 0))],
        out_specs=pl.BlockSpec((R, D16), lambda i, f: (f * nblk + i, 0)),
        out_shape=jax.ShapeDtypeStruct((4 * npad, D16), f32),
    )(x2, Wp2.reshape(64, 4, D16).transpose(1, 0, 2),
      bp2r.reshape(1, 4, D16).transpose(1, 0, 2))

    s2 = sc2(srcp, dstp, ewp, xp2q, zrows)

    x3, y3t = pl.pallas_call(
        _tc_c,
        grid=(nblk,),
        in_specs=[_row_spec(0), _row_spec(nblk), _row_spec(2 * nblk),
                  _row_spec(3 * nblk), _row_spec(0), _row_spec(nblk),
                  pl.BlockSpec((R, 64), lambda i: (i, 0)),
                  _full((64, 64)), _full((1, 64)), _full((64, 64)),
                  _full((64, 64)), _full((1, 64)), _full((64, 5))],
        out_specs=[pl.BlockSpec((R, 64), lambda i: (i, 0)), _row_spec()],
        out_shape=[jax.ShapeDtypeStruct((npad, 64), f32),
                   jax.ShapeDtypeStruct((npad, D16), f32)],
    )(s2, s2, s2, s2, s1, s1, x2, Wl2, bl2r, Wr2, Wp3, bp3r, Wl3)

    s3 = sc3(srcp, dstp, ewp, y3t, zrows)

    out = pl.pallas_call(
        _tc_d,
        grid=(nblk,),
        in_specs=[_row_spec(0), _row_spec(nblk), _row_spec(0),
                  _row_spec(nblk),
                  pl.BlockSpec((R, 64), lambda i: (i, 0)),
                  _full((64, 5)), _full((1, 5))],
        out_specs=pl.BlockSpec((R, 5), lambda i: (i, 0)),
        out_shape=jax.ShapeDtypeStruct((n, 5), f32),
    )(s3, s3, s1, s1, x3, Wr3, bl3r)

    return out


# speedup vs baseline: 10.7713x; 1.0784x over previous
import functools

import jax
import jax.numpy as jnp
from jax import lax
from jax.experimental import pallas as pl
from jax.experimental.pallas import tpu as pltpu
from jax.experimental.pallas import tpu_sc as plsc

NC = 2
NS = 16
L = 16

CH = 512
SUB = 128
NSUB = CH // SUB
R = 2048
D16 = 16


def _pad_rows(a, rows):
    return jnp.concatenate(
        [a, jnp.zeros((rows - a.shape[0],) + a.shape[1:], a.dtype)], axis=0)


def _bn(h, g0, b0):
    return (h * (1.0 / (1.0 + 1e-5) ** 0.5)) * g0 + b0


def _norm_div(o):
    nrm = jnp.maximum(jnp.sqrt(jnp.sum(o * o, axis=1, keepdims=True)), 1e-12)
    return o / nrm


def _tc_a(h, g0, b0, wp1, bp1, xp1t):
    x = _bn(h[...], g0[...], b0[...])
    xp = jax.nn.relu(jnp.dot(x, wp1[...], preferred_element_type=jnp.float32)
                     + bp1[...])
    one = jnp.ones((R, 1), jnp.float32)
    zer = jnp.zeros((R, D16 - 6), jnp.float32)
    xp1t[...] = jnp.concatenate([xp, one, zer], axis=1)


def _tc_b(h, s1a, s1b, g0, b0, wl1, bl1, wr1, x2o):
    x = _bn(h[...], g0[...], b0[...])
    s = s1a[...] + s1b[...]
    cntc = jnp.maximum(s[:, 5:6], 1.0)
    agg = s[:, 0:5] / cntc
    o = (jnp.dot(agg, wl1[...], preferred_element_type=jnp.float32) + bl1[...]
         + jnp.dot(x, wr1[...], preferred_element_type=jnp.float32))
    x2o[...] = jax.nn.relu(_norm_div(o))


def _tc_b2(x2, wp2q, bp2q, xp2qo):
    xp2qo[...] = jax.nn.relu(
        jnp.dot(x2[...], wp2q[0], preferred_element_type=jnp.float32)
        + bp2q[0])


def _tc_c(s2q0, s2q1, s2q2, s2q3, s1a, s1b, x2, wl2, bl2, wr2, wp3, bp3, wl3,
          x3o, y3o):
    wl2v = wl2[...]
    acc = jnp.dot(s2q0[...], wl2v[0:16, :], preferred_element_type=jnp.float32)
    acc += jnp.dot(s2q1[...], wl2v[16:32, :], preferred_element_type=jnp.float32)
    acc += jnp.dot(s2q2[...], wl2v[32:48, :], preferred_element_type=jnp.float32)
    acc += jnp.dot(s2q3[...], wl2v[48:64, :], preferred_element_type=jnp.float32)
    s = s1a[...] + s1b[...]
    cntc = jnp.maximum(s[:, 5:6], 1.0)
    o = (acc / cntc + bl2[...]
         + jnp.dot(x2[...], wr2[...], preferred_element_type=jnp.float32))
    x3 = jax.nn.relu(_norm_div(o))
    x3o[...] = x3
    xp3 = jax.nn.relu(
        jnp.dot(x3, wp3[...], preferred_element_type=jnp.float32) + bp3[...])
    y3 = jnp.dot(xp3, wl3[...], preferred_element_type=jnp.float32)
    y3o[...] = jnp.concatenate(
        [y3, jnp.zeros((R, D16 - 5), jnp.float32)], axis=1)


def _tc_d(s3a, s3b, s1a, s1b, x3, wr3, bl3, out):
    s3 = s3a[...] + s3b[...]
    s = s1a[...] + s1b[...]
    cntc = jnp.maximum(s[:, 5:6], 1.0)
    agg = s3[:, 0:5] / cntc
    o = (agg + bl3[...]
         + jnp.dot(x3[...], wr3[...], preferred_element_type=jnp.float32))
    out[...] = _norm_div(o)


def _row_spec(off_blocks=0):
    return pl.BlockSpec((R, D16), lambda i, o=off_blocks: (o + i, 0))


def _full(shape):
    return pl.BlockSpec(shape, lambda i: tuple(0 for _ in shape))


_GDN = lax.GatherDimensionNumbers(
    offset_dims=(), collapsed_slice_dims=(0,), start_index_map=(0,))


def _bcast_lane(vec, k):
    idx = jnp.full((L, 1), k, jnp.int32)
    return lax.gather(vec, idx, _GDN, (1,),
                      mode=lax.GatherScatterMode.PROMISE_IN_BOUNDS)


def _scale_chunk(g_ref, ew_ref, with_cnt):
    lane = lax.iota(jnp.int32, L)

    def group(g, _):
        base = g * L
        wv = ew_ref[pl.ds(base, L)]

        for k in range(L):
            j = base + k
            w = _bcast_lane(wv, k)
            if with_cnt:
                w = jnp.where(lane == 5, 1.0, w)
            g_ref[j] = g_ref[j] * w
        return ()

    lax.fori_loop(0, CH // L, group, (), unroll=2)


def _pipelined_accumulate(srcp, dstp, ewp, tab, sbuf, dbuf, gidx, sidx, ewb,
                          g, acc, esems, gsems, ssems, cbase, n_chunks, q,
                          with_cnt):
    def start_edges(ci, b):
        off = ci * CH
        pltpu.async_copy(srcp.at[pl.ds(off, CH)], sbuf.at[b], esems[b])
        pltpu.async_copy(ewp.at[pl.ds(off, CH)], ewb.at[b], esems[b])
        pltpu.async_copy(dstp.at[pl.ds(off, CH)], dbuf.at[b], esems[b])

    def edges_wait(b):
        pltpu.make_async_copy(srcp.at[pl.ds(0, CH)], sbuf.at[b],
                              esems[b]).wait()
        pltpu.make_async_copy(ewp.at[pl.ds(0, CH)], ewb.at[b],
                              esems[b]).wait()
        pltpu.make_async_copy(dstp.at[pl.ds(0, CH)], dbuf.at[b],
                              esems[b]).wait()

    def gather_idx_ref(b, u):
        return gidx.at[b, u] if q is not None else sbuf.at[b, pl.ds(u * SUB,
                                                                    SUB)]

    def stage_and_fire_gathers(b):
        for u in range(NSUB):
            sr = sidx.at[b, u]
            for k in range(SUB // L):
                sr[pl.ds(k * L, L)] = dbuf.at[b][pl.ds(u * SUB + k * L, L)]
        if q is not None:
            for u in range(NSUB):
                gr = gidx.at[b, u]
                for k in range(SUB // L):
                    gr[pl.ds(k * L, L)] = (
                        sbuf.at[b][pl.ds(u * SUB + k * L, L)] + q)
        for u in range(NSUB):
            pltpu.async_copy(tab.at[gather_idx_ref(b, u)],
                             g.at[b, pl.ds(u * SUB, SUB), :], gsems[b])

    def gathers_wait(b):
        for u in range(NSUB):
            pltpu.make_async_copy(tab.at[gather_idx_ref(b, u)],
                                  g.at[b, pl.ds(u * SUB, SUB), :],
                                  gsems[b]).wait()

    def scatter_wait(b):
        for u in range(NSUB):
            pltpu.make_async_copy(g.at[b, pl.ds(u * SUB, SUB), :],
                                  acc.at[sidx.at[b, u]], ssems[b]).wait()

    start_edges(cbase, 0)
    edges_wait(0)
    stage_and_fire_gathers(0)
    start_edges(cbase + 1, 1)

    def pair(ii, _):
        for b in range(2):
            i = ii * 2 + b
            gathers_wait(b)
            _scale_chunk(g.at[b], ewb.at[b], with_cnt)
            for u in range(NSUB):
                pltpu.async_copy(g.at[b, pl.ds(u * SUB, SUB), :],
                                 acc.at[sidx.at[b, u]], ssems[b], add=True)
            edges_wait(b ^ 1)
            @pl.when(i >= 1)
            def _():
                scatter_wait(b ^ 1)
            stage_and_fire_gathers(b ^ 1)
            start_edges(cbase + i + 2, b)
        return ()

    lax.fori_loop(0, n_chunks // 2, pair, ())
    gathers_wait(n_chunks % 2)
    scatter_wait((n_chunks - 1) % 2)
    edges_wait((n_chunks - 1) % 2)


def _sc_edge_pass_13(nacc, rpt, n_chunks, with_cnt):
    def body(srcp, dstp, ewp, tab_h, z_h, out_h, sbuf, dbuf, sidx, ewb, g,
             acc, esem0, esem1, gsem0, gsem1, ssem0, ssem1):
        c = lax.axis_index("c")
        s = lax.axis_index("s")
        row0 = s * rpt
        pltpu.sync_copy(z_h, acc.at[pl.ds(row0, rpt)])
        plsc.subcore_barrier()
        cbase = (c * NS + s) * n_chunks
        _pipelined_accumulate(srcp, dstp, ewp, tab_h, sbuf, dbuf, None, sidx,
                              ewb, g, acc, (esem0, esem1), (gsem0, gsem1),
                              (ssem0, ssem1), cbase, n_chunks, None, with_cnt)
        plsc.subcore_barrier()
        pltpu.sync_copy(acc.at[pl.ds(row0, rpt)],
                        out_h.at[pl.ds(c * nacc + row0, rpt)])

    return body


def _sc_edge_pass_2(nacc, rpt, n_chunks):
    def body(srcp, dstp, ewp, tab_h, z_h, out_h, sbuf, dbuf, gidx, sidx, ewb,
             g, acc, esem0, esem1, gsem0, gsem1, ssem0, ssem1):
        c = lax.axis_index("c")
        s = lax.axis_index("s")
        row0 = s * rpt
        cbase = s * n_chunks

        for p in range(2):
            q = c * 2 + p
            pltpu.sync_copy(z_h, acc.at[pl.ds(row0, rpt)])
            plsc.subcore_barrier()
            _pipelined_accumulate(srcp, dstp, ewp, tab_h, sbuf, dbuf, gidx,
                                  sidx, ewb, g, acc, (esem0, esem1),
                                  (gsem0, gsem1), (ssem0, ssem1), cbase,
                                  n_chunks, q * nacc, False)
            plsc.subcore_barrier()
            pltpu.sync_copy(acc.at[pl.ds(row0, rpt)],
                            out_h.at[pl.ds(q * nacc + row0, rpt)])
            plsc.subcore_barrier()

    return body


def kernel(h, edge_index, edge_weight, g0, b0, Wp1, bp1, Wl1, bl1, Wr1,
           Wp2, bp2, Wl2, bl2, Wr2, Wp3, bp3, Wl3, bl3, Wr3):
    n = h.shape[0]
    e = edge_weight.shape[0]
    f32 = jnp.float32

    npad = -(-n // R) * R
    rpt = npad // NS
    dump = npad - n
    nblk = npad // R

    epad = -(-e // (NC * NS * CH)) * (NC * NS * CH)
    nch13 = epad // (NC * NS * CH)
    nch2 = epad // (NS * CH)
    pe = epad + 2 * CH - e
    pidx = lax.iota(jnp.int32, pe)
    srcp = jnp.concatenate([edge_index[0], pidx % n])
    dstp = jnp.concatenate([edge_index[1], n + pidx % max(dump, 1)])
    ewp = jnp.concatenate([edge_weight, jnp.zeros((pe,), f32)])

    zrows = jnp.zeros((rpt, D16), f32)
    g0r, b0r = g0.reshape(1, 5), b0.reshape(1, 5)
    bp1r, bl1r = bp1.reshape(1, 5), bl1.reshape(1, 64)
    bp2r, bl2r = bp2.reshape(1, 64), bl2.reshape(1, 64)
    bp3r, bl3r = bp3.reshape(1, 64), bl3.reshape(1, 5)

    mesh = plsc.VectorSubcoreMesh(core_axis_name="c", subcore_axis_name="s",
                                  num_cores=NC, num_subcores=NS)
    sems = [pltpu.SemaphoreType.DMA] * 6
    common = [
        pltpu.VMEM((2, NSUB, SUB), jnp.int32),
        pltpu.VMEM((2, CH), f32),
        pltpu.VMEM((2, CH, D16), f32),
        pltpu.VMEM_SHARED((npad, D16), f32),
    ]
    sc13_scratch = ([pltpu.VMEM((2, CH), jnp.int32),
                     pltpu.VMEM((2, CH), jnp.int32)]
                    + common + sems)
    sc2_scratch = ([pltpu.VMEM((2, CH), jnp.int32),
                    pltpu.VMEM((2, CH), jnp.int32),
                    pltpu.VMEM((2, NSUB, SUB), jnp.int32)]
                   + common + sems)

    sc_params = pltpu.CompilerParams(use_tc_tiling_on_sc=False)
    sc1 = pl.kernel(_sc_edge_pass_13(npad, rpt, nch13, True),
                    out_type=jax.ShapeDtypeStruct((2 * npad, D16), f32),
                    mesh=mesh, scratch_types=sc13_scratch,
                    compiler_params=sc_params)
    sc3 = pl.kernel(_sc_edge_pass_13(npad, rpt, nch13, False),
                    out_type=jax.ShapeDtypeStruct((2 * npad, D16), f32),
                    mesh=mesh, scratch_types=sc13_scratch,
                    compiler_params=sc_params)
    sc2 = pl.kernel(_sc_edge_pass_2(npad, rpt, nch2),
                    out_type=jax.ShapeDtypeStruct((4 * npad, D16), f32),
                    mesh=mesh, scratch_types=sc2_scratch,
                    compiler_params=sc_params)

    xp1t = pl.pallas_call(
        _tc_a,
        grid=(nblk,),
        in_specs=[pl.BlockSpec((R, 5), lambda i: (i, 0)), _full((1, 5)),
                  _full((1, 5)), _full((5, 5)), _full((1, 5))],
        out_specs=_row_spec(),
        out_shape=jax.ShapeDtypeStruct((npad, D16), f32),
    )(h, g0r, b0r, Wp1, bp1r)

    s1 = sc1(srcp, dstp, ewp, xp1t, zrows)

    x2 = pl.pallas_call(
        _tc_b,
        grid=(nblk,),
        in_specs=[pl.BlockSpec((R, 5), lambda i: (i, 0)),
                  _row_spec(0), _row_spec(nblk),
                  _full((1, 5)), _full((1, 5)), _full((5, 64)),
                  _full((1, 64)), _full((5, 64))],
        out_specs=pl.BlockSpec((R, 64), lambda i: (i, 0)),
        out_shape=jax.ShapeDtypeStruct((npad, 64), f32),
    )(h, s1, s1, g0r, b0r, Wl1, bl1r, Wr1)

    xp2q = pl.pallas_call(
        _tc_b2,
        grid=(nblk, 4),
        in_specs=[pl.BlockSpec((R, 64), lambda i, f: (i, 0)),
                  pl.BlockSpec((1, 64, D16), lambda i, f: (f, 0, 0)),
                  pl.BlockSpec((1, 1, D16), lambda i, f: (f, 0, 0))],
        out_specs=pl.BlockSpec((R, D16), lambda i, f: (f * nblk + i, 0)),
        out_shape=jax.ShapeDtypeStruct((4 * npad, D16), f32),
    )(x2, Wp2.reshape(64, 4, D16).transpose(1, 0, 2),
      bp2r.reshape(1, 4, D16).transpose(1, 0, 2))

    s2 = sc2(srcp, dstp, ewp, xp2q, zrows)

    x3, y3t = pl.pallas_call(
        _tc_c,
        grid=(nblk,),
        in_specs=[_row_spec(0), _row_spec(nblk), _row_spec(2 * nblk),
                  _row_spec(3 * nblk), _row_spec(0), _row_spec(nblk),
                  pl.BlockSpec((R, 64), lambda i: (i, 0)),
                  _full((64, 64)), _full((1, 64)), _full((64, 64)),
                  _full((64, 64)), _full((1, 64)), _full((64, 5))],
        out_specs=[pl.BlockSpec((R, 64), lambda i: (i, 0)), _row_spec()],
        out_shape=[jax.ShapeDtypeStruct((npad, 64), f32),
                   jax.ShapeDtypeStruct((npad, D16), f32)],
    )(s2, s2, s2, s2, s1, s1, x2, Wl2, bl2r, Wr2, Wp3, bp3r, Wl3)

    s3 = sc3(srcp, dstp, ewp, y3t, zrows)

    out = pl.pallas_call(
        _tc_d,
        grid=(nblk,),
        in_specs=[_row_spec(0), _row_spec(nblk), _row_spec(0),
                  _row_spec(nblk),
                  pl.BlockSpec((R, 64), lambda i: (i, 0)),
                  _full((64, 5)), _full((1, 5))],
        out_specs=pl.BlockSpec((R, 5), lambda i: (i, 0)),
        out_shape=jax.ShapeDtypeStruct((n, 5), f32),
    )(s3, s3, s1, s1, x3, Wr3, bl3r)

    return out


# speedup vs baseline: 11.7185x; 1.0879x over previous
import functools

import jax
import jax.numpy as jnp
from jax import lax
from jax.experimental import pallas as pl
from jax.experimental.pallas import tpu as pltpu
from jax.experimental.pallas import tpu_sc as plsc

NC = 2
NS = 16
L = 16

CH = 512
SUB = 128
NSUB = CH // SUB
R = 2048
D16 = 16


def _pad_rows(a, rows):
    return jnp.concatenate(
        [a, jnp.zeros((rows - a.shape[0],) + a.shape[1:], a.dtype)], axis=0)


def _bn(h, g0, b0):
    return (h * (1.0 / (1.0 + 1e-5) ** 0.5)) * g0 + b0


def _norm_div(o):
    nrm = jnp.maximum(jnp.sqrt(jnp.sum(o * o, axis=1, keepdims=True)), 1e-12)
    return o / nrm


def _tc_a(h, g0, b0, wp1, bp1, xp1t):
    x = _bn(h[...], g0[...], b0[...])
    xp = jax.nn.relu(jnp.dot(x, wp1[...], preferred_element_type=jnp.float32)
                     + bp1[...])
    one = jnp.ones((R, 1), jnp.float32)
    zer = jnp.zeros((R, 128 - 6), jnp.float32)
    xp1t[...] = jnp.concatenate([xp, one, zer], axis=1)


def _tc_b(h, s1a, s1b, g0, b0, wl1, bl1, wr1, x2o):
    x = _bn(h[...], g0[...], b0[...])
    s = s1a[...] + s1b[...]
    cntc = jnp.maximum(s[:, 5:6], 1.0)
    agg = s[:, 0:5] / cntc
    o = (jnp.dot(agg, wl1[...], preferred_element_type=jnp.float32) + bl1[...]
         + jnp.dot(x, wr1[...], preferred_element_type=jnp.float32))
    x2o[...] = jax.nn.relu(_norm_div(o))


def _tc_b2(x2, wp2q, bp2q, xp2qo):
    xq = jax.nn.relu(
        jnp.dot(x2[...], wp2q[0], preferred_element_type=jnp.float32)
        + bp2q[0])
    xp2qo[...] = jnp.concatenate(
        [xq, jnp.zeros((R, 128 - D16), jnp.float32)], axis=1)


def _tc_c(s2q0, s2q1, s2q2, s2q3, s1a, s1b, x2, wl2, bl2, wr2, wp3, bp3, wl3,
          x3o, y3o):
    wl2v = wl2[...]
    acc = jnp.dot(s2q0[...], wl2v[0:16, :], preferred_element_type=jnp.float32)
    acc += jnp.dot(s2q1[...], wl2v[16:32, :], preferred_element_type=jnp.float32)
    acc += jnp.dot(s2q2[...], wl2v[32:48, :], preferred_element_type=jnp.float32)
    acc += jnp.dot(s2q3[...], wl2v[48:64, :], preferred_element_type=jnp.float32)
    s = s1a[...] + s1b[...]
    cntc = jnp.maximum(s[:, 5:6], 1.0)
    o = (acc / cntc + bl2[...]
         + jnp.dot(x2[...], wr2[...], preferred_element_type=jnp.float32))
    x3 = jax.nn.relu(_norm_div(o))
    x3o[...] = x3
    xp3 = jax.nn.relu(
        jnp.dot(x3, wp3[...], preferred_element_type=jnp.float32) + bp3[...])
    y3 = jnp.dot(xp3, wl3[...], preferred_element_type=jnp.float32)
    y3o[...] = jnp.concatenate(
        [y3, jnp.zeros((R, 128 - 5), jnp.float32)], axis=1)


def _tc_d(s3a, s3b, s1a, s1b, x3, wr3, bl3, out):
    s3 = s3a[...] + s3b[...]
    s = s1a[...] + s1b[...]
    cntc = jnp.maximum(s[:, 5:6], 1.0)
    agg = s3[:, 0:5] / cntc
    o = (agg + bl3[...]
         + jnp.dot(x3[...], wr3[...], preferred_element_type=jnp.float32))
    out[...] = _norm_div(o)


def _row_spec(off_blocks=0):
    return pl.BlockSpec((R, D16), lambda i, o=off_blocks: (o + i, 0))


def _full(shape):
    return pl.BlockSpec(shape, lambda i: tuple(0 for _ in shape))


_GDN = lax.GatherDimensionNumbers(
    offset_dims=(), collapsed_slice_dims=(0,), start_index_map=(0,))


def _bcast_lane(vec, k):
    idx = jnp.full((L, 1), k, jnp.int32)
    return lax.gather(vec, idx, _GDN, (1,),
                      mode=lax.GatherScatterMode.PROMISE_IN_BOUNDS)


def _scale_chunk(g_ref, ew_ref, with_cnt):
    lane = lax.iota(jnp.int32, L)

    def group(g, _):
        base = g * L
        wv = ew_ref[pl.ds(base, L)]

        for k in range(L):
            j = base + k
            w = _bcast_lane(wv, k)
            if with_cnt:
                w = jnp.where(lane == 5, 1.0, w)
            g_ref[j] = g_ref[j] * w
        return ()

    lax.fori_loop(0, CH // L, group, (), unroll=2)


def _pipelined_accumulate(srcp, dstp, ewp, tab, sbuf, dbuf, gidx, sidx, ewb,
                          g, acc, esems, gsems, ssems, cbase, n_chunks, q,
                          with_cnt):
    def start_edges(ci, b):
        off = ci * CH
        pltpu.async_copy(srcp.at[pl.ds(off, CH)], sbuf.at[b], esems[b])
        pltpu.async_copy(ewp.at[pl.ds(off, CH)], ewb.at[b], esems[b])
        pltpu.async_copy(dstp.at[pl.ds(off, CH)], dbuf.at[b], esems[b])

    def edges_wait(b):
        pltpu.make_async_copy(srcp.at[pl.ds(0, CH)], sbuf.at[b],
                              esems[b]).wait()
        pltpu.make_async_copy(ewp.at[pl.ds(0, CH)], ewb.at[b],
                              esems[b]).wait()
        pltpu.make_async_copy(dstp.at[pl.ds(0, CH)], dbuf.at[b],
                              esems[b]).wait()

    def gather_idx_ref(b, u):
        return gidx.at[b, u]

    def stage_and_fire_gathers(b):
        for u in range(NSUB):
            sr = sidx.at[b, u]
            for k in range(SUB // L):
                sr[pl.ds(k * L, L)] = dbuf.at[b][pl.ds(u * SUB + k * L, L)]
        for u in range(NSUB):
            gr = gidx.at[b, u]
            for k in range(SUB // L):
                gr[pl.ds(k * L, L)] = (
                    sbuf.at[b][pl.ds(u * SUB + k * L, L)] * 8 + q)
        for u in range(NSUB):
            pltpu.async_copy(tab.at[gather_idx_ref(b, u)],
                             g.at[b, pl.ds(u * SUB, SUB), :], gsems[b])

    def gathers_wait(b):
        for u in range(NSUB):
            pltpu.make_async_copy(tab.at[gather_idx_ref(b, u)],
                                  g.at[b, pl.ds(u * SUB, SUB), :],
                                  gsems[b]).wait()

    def scatter_wait(b):
        for u in range(NSUB):
            pltpu.make_async_copy(g.at[b, pl.ds(u * SUB, SUB), :],
                                  acc.at[sidx.at[b, u]], ssems[b]).wait()

    start_edges(cbase, 0)
    edges_wait(0)
    stage_and_fire_gathers(0)
    start_edges(cbase + 1, 1)

    def pair(ii, _):
        for b in range(2):
            i = ii * 2 + b
            gathers_wait(b)
            _scale_chunk(g.at[b], ewb.at[b], with_cnt)
            for u in range(NSUB):
                pltpu.async_copy(g.at[b, pl.ds(u * SUB, SUB), :],
                                 acc.at[sidx.at[b, u]], ssems[b], add=True)
            edges_wait(b ^ 1)
            @pl.when(i >= 1)
            def _():
                scatter_wait(b ^ 1)
            stage_and_fire_gathers(b ^ 1)
            start_edges(cbase + i + 2, b)
        return ()

    lax.fori_loop(0, n_chunks // 2, pair, ())
    gathers_wait(n_chunks % 2)
    scatter_wait((n_chunks - 1) % 2)
    edges_wait((n_chunks - 1) % 2)


def _sc_edge_pass_13(nacc, rpt, n_chunks, with_cnt):
    def body(srcp, dstp, ewp, tab_h, z_h, out_h, sbuf, dbuf, gidx, sidx, ewb,
             g, acc, esem0, esem1, gsem0, gsem1, ssem0, ssem1):
        c = lax.axis_index("c")
        s = lax.axis_index("s")
        row0 = s * rpt
        pltpu.sync_copy(z_h, acc.at[pl.ds(row0, rpt)])
        plsc.subcore_barrier()
        cbase = (c * NS + s) * n_chunks
        _pipelined_accumulate(srcp, dstp, ewp, tab_h, sbuf, dbuf, gidx, sidx,
                              ewb, g, acc, (esem0, esem1), (gsem0, gsem1),
                              (ssem0, ssem1), cbase, n_chunks, 0, with_cnt)
        plsc.subcore_barrier()
        pltpu.sync_copy(acc.at[pl.ds(row0, rpt)],
                        out_h.at[pl.ds(c * nacc + row0, rpt)])

    return body


def _sc_edge_pass_2(nacc, rpt, n_chunks):
    def body(srcp, dstp, ewp, tab_h, z_h, out_h, sbuf, dbuf, gidx, sidx, ewb,
             g, acc, esem0, esem1, gsem0, gsem1, ssem0, ssem1):
        c = lax.axis_index("c")
        s = lax.axis_index("s")
        row0 = s * rpt
        cbase = s * n_chunks

        for p in range(2):
            q = c * 2 + p
            pltpu.sync_copy(z_h, acc.at[pl.ds(row0, rpt)])
            plsc.subcore_barrier()
            _pipelined_accumulate(srcp, dstp, ewp, tab_h, sbuf, dbuf, gidx,
                                  sidx, ewb, g, acc, (esem0, esem1),
                                  (gsem0, gsem1), (ssem0, ssem1), cbase,
                                  n_chunks, q * (8 * nacc), False)
            plsc.subcore_barrier()
            pltpu.sync_copy(acc.at[pl.ds(row0, rpt)],
                            out_h.at[pl.ds(q * nacc + row0, rpt)])
            plsc.subcore_barrier()

    return body


def kernel(h, edge_index, edge_weight, g0, b0, Wp1, bp1, Wl1, bl1, Wr1,
           Wp2, bp2, Wl2, bl2, Wr2, Wp3, bp3, Wl3, bl3, Wr3):
    n = h.shape[0]
    e = edge_weight.shape[0]
    f32 = jnp.float32

    npad = -(-n // R) * R
    rpt = npad // NS
    dump = npad - n
    nblk = npad // R

    epad = -(-e // (NC * NS * CH)) * (NC * NS * CH)
    nch13 = epad // (NC * NS * CH)
    nch2 = epad // (NS * CH)
    pe = epad + 2 * CH - e
    pidx = lax.iota(jnp.int32, pe)
    srcp = jnp.concatenate([edge_index[0], pidx % n])
    dstp = jnp.concatenate([edge_index[1], n + pidx % max(dump, 1)])
    ewp = jnp.concatenate([edge_weight, jnp.zeros((pe,), f32)])

    zrows = jnp.zeros((rpt, D16), f32)
    g0r, b0r = g0.reshape(1, 5), b0.reshape(1, 5)
    bp1r, bl1r = bp1.reshape(1, 5), bl1.reshape(1, 64)
    bp2r, bl2r = bp2.reshape(1, 64), bl2.reshape(1, 64)
    bp3r, bl3r = bp3.reshape(1, 64), bl3.reshape(1, 5)

    mesh = plsc.VectorSubcoreMesh(core_axis_name="c", subcore_axis_name="s",
                                  num_cores=NC, num_subcores=NS)
    sems = [pltpu.SemaphoreType.DMA] * 6
    common = [
        pltpu.VMEM((2, NSUB, SUB), jnp.int32),
        pltpu.VMEM((2, CH), f32),
        pltpu.VMEM((2, CH, D16), f32),
        pltpu.VMEM_SHARED((npad, D16), f32),
    ]
    edgebufs = [pltpu.VMEM((2, CH), jnp.int32),
                pltpu.VMEM((2, CH), jnp.int32),
                pltpu.VMEM((2, NSUB, SUB), jnp.int32)]
    sc13_scratch = edgebufs + common + sems
    sc2_scratch = edgebufs + common + sems

    sc_params = pltpu.CompilerParams(use_tc_tiling_on_sc=False)
    sc1 = pl.kernel(_sc_edge_pass_13(npad, rpt, nch13, True),
                    out_type=jax.ShapeDtypeStruct((2 * npad, D16), f32),
                    mesh=mesh, scratch_types=sc13_scratch,
                    compiler_params=sc_params)
    sc3 = pl.kernel(_sc_edge_pass_13(npad, rpt, nch13, False),
                    out_type=jax.ShapeDtypeStruct((2 * npad, D16), f32),
                    mesh=mesh, scratch_types=sc13_scratch,
                    compiler_params=sc_params)
    sc2 = pl.kernel(_sc_edge_pass_2(npad, rpt, nch2),
                    out_type=jax.ShapeDtypeStruct((4 * npad, D16), f32),
                    mesh=mesh, scratch_types=sc2_scratch,
                    compiler_params=sc_params)

    xp1t = pl.pallas_call(
        _tc_a,
        grid=(nblk,),
        in_specs=[pl.BlockSpec((R, 5), lambda i: (i, 0)), _full((1, 5)),
                  _full((1, 5)), _full((5, 5)), _full((1, 5))],
        out_specs=pl.BlockSpec((R, 128), lambda i: (i, 0)),
        out_shape=jax.ShapeDtypeStruct((npad, 128), f32),
    )(h, g0r, b0r, Wp1, bp1r)

    s1 = sc1(srcp, dstp, ewp, xp1t.reshape(8 * npad, D16), zrows)

    x2 = pl.pallas_call(
        _tc_b,
        grid=(nblk,),
        in_specs=[pl.BlockSpec((R, 5), lambda i: (i, 0)),
                  _row_spec(0), _row_spec(nblk),
                  _full((1, 5)), _full((1, 5)), _full((5, 64)),
                  _full((1, 64)), _full((5, 64))],
        out_specs=pl.BlockSpec((R, 64), lambda i: (i, 0)),
        out_shape=jax.ShapeDtypeStruct((npad, 64), f32),
    )(h, s1, s1, g0r, b0r, Wl1, bl1r, Wr1)

    xp2q = pl.pallas_call(
        _tc_b2,
        grid=(nblk, 4),
        in_specs=[pl.BlockSpec((R, 64), lambda i, f: (i, 0)),
                  pl.BlockSpec((1, 64, D16), lambda i, f: (f, 0, 0)),
                  pl.BlockSpec((1, 1, D16), lambda i, f: (f, 0, 0))],
        out_specs=pl.BlockSpec((R, 128), lambda i, f: (f * nblk + i, 0)),
        out_shape=jax.ShapeDtypeStruct((4 * npad, 128), f32),
    )(x2, Wp2.reshape(64, 4, D16).transpose(1, 0, 2),
      bp2r.reshape(1, 4, D16).transpose(1, 0, 2))

    s2 = sc2(srcp, dstp, ewp, xp2q.reshape(32 * npad, D16), zrows)

    x3, y3t = pl.pallas_call(
        _tc_c,
        grid=(nblk,),
        in_specs=[_row_spec(0), _row_spec(nblk), _row_spec(2 * nblk),
                  _row_spec(3 * nblk), _row_spec(0), _row_spec(nblk),
                  pl.BlockSpec((R, 64), lambda i: (i, 0)),
                  _full((64, 64)), _full((1, 64)), _full((64, 64)),
                  _full((64, 64)), _full((1, 64)), _full((64, 5))],
        out_specs=[pl.BlockSpec((R, 64), lambda i: (i, 0)),
                   pl.BlockSpec((R, 128), lambda i: (i, 0))],
        out_shape=[jax.ShapeDtypeStruct((npad, 64), f32),
                   jax.ShapeDtypeStruct((npad, 128), f32)],
    )(s2, s2, s2, s2, s1, s1, x2, Wl2, bl2r, Wr2, Wp3, bp3r, Wl3)

    s3 = sc3(srcp, dstp, ewp, y3t.reshape(8 * npad, D16), zrows)

    out = pl.pallas_call(
        _tc_d,
        grid=(nblk,),
        in_specs=[_row_spec(0), _row_spec(nblk), _row_spec(0),
                  _row_spec(nblk),
                  pl.BlockSpec((R, 64), lambda i: (i, 0)),
                  _full((64, 5)), _full((1, 5))],
        out_specs=pl.BlockSpec((R, 5), lambda i: (i, 0)),
        out_shape=jax.ShapeDtypeStruct((n, 5), f32),
    )(s3, s3, s1, s1, x3, Wr3, bl3r)

    return out


# speedup vs baseline: 11.8973x; 1.0153x over previous
import functools

import jax
import jax.numpy as jnp
from jax import lax
from jax.experimental import pallas as pl
from jax.experimental.pallas import tpu as pltpu
from jax.experimental.pallas import tpu_sc as plsc

NC = 2
NS = 16
L = 16

CH = 512
SUB = 128
NSUB = CH // SUB
R = 2048
D16 = 16


def _pad_rows(a, rows):
    return jnp.concatenate(
        [a, jnp.zeros((rows - a.shape[0],) + a.shape[1:], a.dtype)], axis=0)


def _bn(h, g0, b0):
    return (h * (1.0 / (1.0 + 1e-5) ** 0.5)) * g0 + b0


def _norm_div(o):
    nrm = jnp.maximum(jnp.sqrt(jnp.sum(o * o, axis=1, keepdims=True)), 1e-12)
    return o / nrm


def _tc_a(h, g0, b0, wp1, bp1, xp1t):
    x = _bn(h[...], g0[...], b0[...])
    xp = jax.nn.relu(jnp.dot(x, wp1[...], preferred_element_type=jnp.float32)
                     + bp1[...])
    one = jnp.ones((R, 1), jnp.float32)
    zer = jnp.zeros((R, 128 - 6), jnp.float32)
    xp1t[...] = jnp.concatenate([xp, one, zer], axis=1)


def _tc_b(h, s1a, s1b, g0, b0, wl1, bl1, wr1, x2o):
    x = _bn(h[...], g0[...], b0[...])
    s = s1a[...] + s1b[...]
    cntc = jnp.maximum(s[:, 5:6], 1.0)
    agg = s[:, 0:5] / cntc
    o = (jnp.dot(agg, wl1[...], preferred_element_type=jnp.float32) + bl1[...]
         + jnp.dot(x, wr1[...], preferred_element_type=jnp.float32))
    x2o[...] = jnp.concatenate(
        [jax.nn.relu(_norm_div(o)), 1.0 / cntc,
         jnp.zeros((R, 63), jnp.float32)], axis=1)


def _tc_b2(x2, wp2q, bp2q, xp2qo):
    xq = jax.nn.relu(
        jnp.dot(x2[...][:, 0:64], wp2q[0],
                preferred_element_type=jnp.float32)
        + bp2q[0])
    xp2qo[...] = jnp.concatenate(
        [xq, jnp.zeros((R, 128 - D16), jnp.float32)], axis=1)


def _tc_c(s2q0, s2q1, s2q2, s2q3, x2e, wl2, bl2, wr2, wp3, bp3, wl3,
          x3o, y3o):
    wl2v = wl2[...]
    acc = jnp.dot(s2q0[...], wl2v[0:16, :], preferred_element_type=jnp.float32)
    acc += jnp.dot(s2q1[...], wl2v[16:32, :], preferred_element_type=jnp.float32)
    acc += jnp.dot(s2q2[...], wl2v[32:48, :], preferred_element_type=jnp.float32)
    acc += jnp.dot(s2q3[...], wl2v[48:64, :], preferred_element_type=jnp.float32)
    x2eb = x2e[...]
    x2 = x2eb[:, 0:64]
    icnt = x2eb[:, 64:65]
    o = (acc * icnt + bl2[...]
         + jnp.dot(x2, wr2[...], preferred_element_type=jnp.float32))
    x3 = jax.nn.relu(_norm_div(o))
    xp3 = jax.nn.relu(
        jnp.dot(x3, wp3[...], preferred_element_type=jnp.float32) + bp3[...])
    y3 = jnp.dot(xp3, wl3[...], preferred_element_type=jnp.float32)
    x3o[...] = jnp.concatenate(
        [x3, icnt, jnp.zeros((R, 63), jnp.float32)], axis=1)
    y3o[...] = jnp.concatenate(
        [y3, jnp.zeros((R, 128 - 5), jnp.float32)], axis=1)


def _tc_d(s3a, s3b, x3e, wr3, bl3, out):
    s3 = s3a[...] + s3b[...]
    x3eb = x3e[...]
    agg = s3[:, 0:5] * x3eb[:, 64:65]
    o = (agg + bl3[...]
         + jnp.dot(x3eb[:, 0:64], wr3[...],
                   preferred_element_type=jnp.float32))
    out[...] = _norm_div(o)


def _row_spec(off_blocks=0):
    return pl.BlockSpec((R, D16), lambda i, o=off_blocks: (o + i, 0))


def _full(shape):
    return pl.BlockSpec(shape, lambda i: tuple(0 for _ in shape))


_GDN = lax.GatherDimensionNumbers(
    offset_dims=(), collapsed_slice_dims=(0,), start_index_map=(0,))


def _bcast_lane(vec, k):
    idx = jnp.full((L, 1), k, jnp.int32)
    return lax.gather(vec, idx, _GDN, (1,),
                      mode=lax.GatherScatterMode.PROMISE_IN_BOUNDS)


def _scale_chunk(g_ref, ew_ref, with_cnt):
    lane = lax.iota(jnp.int32, L)

    def group(g, _):
        base = g * L
        wv = ew_ref[pl.ds(base, L)]

        for k in range(L):
            j = base + k
            w = _bcast_lane(wv, k)
            if with_cnt:
                w = jnp.where(lane == 5, 1.0, w)
            g_ref[j] = g_ref[j] * w
        return ()

    lax.fori_loop(0, CH // L, group, (), unroll=2)


def _pipelined_accumulate(srcp, dstp, ewp, tab, sbuf, dbuf, gidx, sidx, ewb,
                          g, acc, esems, gsems, ssems, cbase, n_chunks, q,
                          with_cnt):
    def start_edges(ci, b):
        off = ci * CH
        pltpu.async_copy(srcp.at[pl.ds(off, CH)], sbuf.at[b], esems[b])
        pltpu.async_copy(ewp.at[pl.ds(off, CH)], ewb.at[b], esems[b])
        pltpu.async_copy(dstp.at[pl.ds(off, CH)], dbuf.at[b], esems[b])

    def edges_wait(b):
        pltpu.make_async_copy(srcp.at[pl.ds(0, CH)], sbuf.at[b],
                              esems[b]).wait()
        pltpu.make_async_copy(ewp.at[pl.ds(0, CH)], ewb.at[b],
                              esems[b]).wait()
        pltpu.make_async_copy(dstp.at[pl.ds(0, CH)], dbuf.at[b],
                              esems[b]).wait()

    def gather_idx_ref(b, u):
        return gidx.at[b, u]

    def stage_and_fire_gathers(b):
        for u in range(NSUB):
            sr = sidx.at[b, u]
            for k in range(SUB // L):
                sr[pl.ds(k * L, L)] = dbuf.at[b][pl.ds(u * SUB + k * L, L)]
        for u in range(NSUB):
            gr = gidx.at[b, u]
            for k in range(SUB // L):
                gr[pl.ds(k * L, L)] = (
                    sbuf.at[b][pl.ds(u * SUB + k * L, L)] * 8 + q)
        for u in range(NSUB):
            pltpu.async_copy(tab.at[gather_idx_ref(b, u)],
                             g.at[b, pl.ds(u * SUB, SUB), :], gsems[b])

    def gathers_wait(b):
        for u in range(NSUB):
            pltpu.make_async_copy(tab.at[gather_idx_ref(b, u)],
                                  g.at[b, pl.ds(u * SUB, SUB), :],
                                  gsems[b]).wait()

    def scatter_wait(b):
        for u in range(NSUB):
            pltpu.make_async_copy(g.at[b, pl.ds(u * SUB, SUB), :],
                                  acc.at[sidx.at[b, u]], ssems[b]).wait()

    start_edges(cbase, 0)
    edges_wait(0)
    stage_and_fire_gathers(0)
    start_edges(cbase + 1, 1)

    def pair(ii, _):
        for b in range(2):
            i = ii * 2 + b
            gathers_wait(b)
            _scale_chunk(g.at[b], ewb.at[b], with_cnt)
            for u in range(NSUB):
                pltpu.async_copy(g.at[b, pl.ds(u * SUB, SUB), :],
                                 acc.at[sidx.at[b, u]], ssems[b], add=True)
            edges_wait(b ^ 1)
            @pl.when(i >= 1)
            def _():
                scatter_wait(b ^ 1)
            stage_and_fire_gathers(b ^ 1)
            start_edges(cbase + i + 2, b)
        return ()

    lax.fori_loop(0, n_chunks // 2, pair, ())
    gathers_wait(n_chunks % 2)
    scatter_wait((n_chunks - 1) % 2)
    edges_wait((n_chunks - 1) % 2)


def _sc_edge_pass_13(nacc, rpt, n_chunks, with_cnt):
    def body(srcp, dstp, ewp, tab_h, z_h, out_h, sbuf, dbuf, gidx, sidx, ewb,
             g, acc, esem0, esem1, gsem0, gsem1, ssem0, ssem1):
        c = lax.axis_index("c")
        s = lax.axis_index("s")
        row0 = s * rpt
        pltpu.sync_copy(z_h, acc.at[pl.ds(row0, rpt)])
        plsc.subcore_barrier()
        cbase = (c * NS + s) * n_chunks
        _pipelined_accumulate(srcp, dstp, ewp, tab_h, sbuf, dbuf, gidx, sidx,
                              ewb, g, acc, (esem0, esem1), (gsem0, gsem1),
                              (ssem0, ssem1), cbase, n_chunks, 0, with_cnt)
        plsc.subcore_barrier()
        pltpu.sync_copy(acc.at[pl.ds(row0, rpt)],
                        out_h.at[pl.ds(c * nacc + row0, rpt)])

    return body


def _sc_edge_pass_2(nacc, rpt, n_chunks):
    def body(srcp, dstp, ewp, tab_h, z_h, out_h, sbuf, dbuf, gidx, sidx, ewb,
             g, acc, esem0, esem1, gsem0, gsem1, ssem0, ssem1):
        c = lax.axis_index("c")
        s = lax.axis_index("s")
        row0 = s * rpt
        cbase = s * n_chunks

        for p in range(2):
            q = c * 2 + p
            pltpu.sync_copy(z_h, acc.at[pl.ds(row0, rpt)])
            plsc.subcore_barrier()
            _pipelined_accumulate(srcp, dstp, ewp, tab_h, sbuf, dbuf, gidx,
                                  sidx, ewb, g, acc, (esem0, esem1),
                                  (gsem0, gsem1), (ssem0, ssem1), cbase,
                                  n_chunks, q * (8 * nacc), False)
            plsc.subcore_barrier()
            pltpu.sync_copy(acc.at[pl.ds(row0, rpt)],
                            out_h.at[pl.ds(q * nacc + row0, rpt)])
            plsc.subcore_barrier()

    return body


def kernel(h, edge_index, edge_weight, g0, b0, Wp1, bp1, Wl1, bl1, Wr1,
           Wp2, bp2, Wl2, bl2, Wr2, Wp3, bp3, Wl3, bl3, Wr3):
    n = h.shape[0]
    e = edge_weight.shape[0]
    f32 = jnp.float32

    npad = -(-n // R) * R
    rpt = npad // NS
    dump = npad - n
    nblk = npad // R

    epad = -(-e // (NC * NS * CH)) * (NC * NS * CH)
    nch13 = epad // (NC * NS * CH)
    nch2 = epad // (NS * CH)
    pe = epad + 2 * CH - e
    pidx = lax.iota(jnp.int32, pe)
    srcp = jnp.concatenate([edge_index[0], pidx % n])
    dstp = jnp.concatenate([edge_index[1], n + pidx % max(dump, 1)])
    ewp = jnp.concatenate([edge_weight, jnp.zeros((pe,), f32)])

    zrows = jnp.zeros((rpt, D16), f32)
    g0r, b0r = g0.reshape(1, 5), b0.reshape(1, 5)
    bp1r, bl1r = bp1.reshape(1, 5), bl1.reshape(1, 64)
    bp2r, bl2r = bp2.reshape(1, 64), bl2.reshape(1, 64)
    bp3r, bl3r = bp3.reshape(1, 64), bl3.reshape(1, 5)

    mesh = plsc.VectorSubcoreMesh(core_axis_name="c", subcore_axis_name="s",
                                  num_cores=NC, num_subcores=NS)
    sems = [pltpu.SemaphoreType.DMA] * 6
    common = [
        pltpu.VMEM((2, NSUB, SUB), jnp.int32),
        pltpu.VMEM((2, CH), f32),
        pltpu.VMEM((2, CH, D16), f32),
        pltpu.VMEM_SHARED((npad, D16), f32),
    ]
    edgebufs = [pltpu.VMEM((2, CH), jnp.int32),
                pltpu.VMEM((2, CH), jnp.int32),
                pltpu.VMEM((2, NSUB, SUB), jnp.int32)]
    sc13_scratch = edgebufs + common + sems
    sc2_scratch = edgebufs + common + sems

    sc_params = pltpu.CompilerParams(use_tc_tiling_on_sc=False)
    sc1 = pl.kernel(_sc_edge_pass_13(npad, rpt, nch13, True),
                    out_type=jax.ShapeDtypeStruct((2 * npad, D16), f32),
                    mesh=mesh, scratch_types=sc13_scratch,
                    compiler_params=sc_params)
    sc3 = pl.kernel(_sc_edge_pass_13(npad, rpt, nch13, False),
                    out_type=jax.ShapeDtypeStruct((2 * npad, D16), f32),
                    mesh=mesh, scratch_types=sc13_scratch,
                    compiler_params=sc_params)
    sc2 = pl.kernel(_sc_edge_pass_2(npad, rpt, nch2),
                    out_type=jax.ShapeDtypeStruct((4 * npad, D16), f32),
                    mesh=mesh, scratch_types=sc2_scratch,
                    compiler_params=sc_params)

    xp1t = pl.pallas_call(
        _tc_a,
        grid=(nblk,),
        in_specs=[pl.BlockSpec((R, 5), lambda i: (i, 0)), _full((1, 5)),
                  _full((1, 5)), _full((5, 5)), _full((1, 5))],
        out_specs=pl.BlockSpec((R, 128), lambda i: (i, 0)),
        out_shape=jax.ShapeDtypeStruct((npad, 128), f32),
    )(h, g0r, b0r, Wp1, bp1r)

    s1 = sc1(srcp, dstp, ewp, xp1t.reshape(8 * npad, D16), zrows)

    x2 = pl.pallas_call(
        _tc_b,
        grid=(nblk,),
        in_specs=[pl.BlockSpec((R, 5), lambda i: (i, 0)),
                  _row_spec(0), _row_spec(nblk),
                  _full((1, 5)), _full((1, 5)), _full((5, 64)),
                  _full((1, 64)), _full((5, 64))],
        out_specs=pl.BlockSpec((R, 128), lambda i: (i, 0)),
        out_shape=jax.ShapeDtypeStruct((npad, 128), f32),
    )(h, s1, s1, g0r, b0r, Wl1, bl1r, Wr1)

    xp2q = pl.pallas_call(
        _tc_b2,
        grid=(nblk, 4),
        in_specs=[pl.BlockSpec((R, 128), lambda i, f: (i, 0)),
                  pl.BlockSpec((1, 64, D16), lambda i, f: (f, 0, 0)),
                  pl.BlockSpec((1, 1, D16), lambda i, f: (f, 0, 0))],
        out_specs=pl.BlockSpec((R, 128), lambda i, f: (f * nblk + i, 0)),
        out_shape=jax.ShapeDtypeStruct((4 * npad, 128), f32),
    )(x2, Wp2.reshape(64, 4, D16).transpose(1, 0, 2),
      bp2r.reshape(1, 4, D16).transpose(1, 0, 2))

    s2 = sc2(srcp, dstp, ewp, xp2q.reshape(32 * npad, D16), zrows)

    x3, y3t = pl.pallas_call(
        _tc_c,
        grid=(nblk,),
        in_specs=[_row_spec(0), _row_spec(nblk), _row_spec(2 * nblk),
                  _row_spec(3 * nblk),
                  pl.BlockSpec((R, 128), lambda i: (i, 0)),
                  _full((64, 64)), _full((1, 64)), _full((64, 64)),
                  _full((64, 64)), _full((1, 64)), _full((64, 5))],
        out_specs=[pl.BlockSpec((R, 128), lambda i: (i, 0)),
                   pl.BlockSpec((R, 128), lambda i: (i, 0))],
        out_shape=[jax.ShapeDtypeStruct((npad, 128), f32),
                   jax.ShapeDtypeStruct((npad, 128), f32)],
    )(s2, s2, s2, s2, x2, Wl2, bl2r, Wr2, Wp3, bp3r, Wl3)

    s3 = sc3(srcp, dstp, ewp, y3t.reshape(8 * npad, D16), zrows)

    out = pl.pallas_call(
        _tc_d,
        grid=(nblk,),
        in_specs=[_row_spec(0), _row_spec(nblk),
                  pl.BlockSpec((R, 128), lambda i: (i, 0)),
                  _full((64, 5)), _full((1, 5))],
        out_specs=pl.BlockSpec((R, 5), lambda i: (i, 0)),
        out_shape=jax.ShapeDtypeStruct((n, 5), f32),
    )(s3, s3, x3, Wr3, bl3r)

    return out
